# baseline (speedup 1.0000x reference)
import functools

import jax
import jax.numpy as jnp
from jax import lax
from jax.experimental import pallas as pl
from jax.experimental.pallas import tpu as pltpu
from jax.experimental.pallas import tpu_sc as plsc

LANES = 128


@functools.cache
def _make_gather(V: int, D: int, SQ: int, ST: int, B: int):
    info = plsc.get_sparse_core_info()
    NC, NS = info.num_cores, info.num_subcores
    NW = NC * NS
    assert B % (NW * 0 + LANES) == 0 and B // LANES == NW
    assert D == 64

    def tiles(S):
        out = []
        for st in range((S + 7) // 8):
            out.append((st, min(8, S - st * 8)))
        return out

    q_tiles = tiles(SQ)
    t_tiles = tiles(ST)
    n_groups = len(q_tiles) + len(t_tiles)

    mesh = plsc.VectorSubcoreMesh(core_axis_name="c", subcore_axis_name="s")

    @functools.partial(
        pl.kernel,
        out_type=(
            jax.ShapeDtypeStruct((SQ, D, B), jnp.float32),
            jax.ShapeDtypeStruct((ST, D, B), jnp.float32),
        ),
        mesh=mesh,
        compiler_params=pltpu.CompilerParams(use_tc_tiling_on_sc=True,
                                             needs_layout_passes=False),
        scratch_types=[
            pltpu.VMEM((n_groups, 8, LANES), jnp.int32),
            pltpu.VMEM((2, LANES, LANES), jnp.float32),
            pltpu.VMEM((2, D, LANES), jnp.float32),
            pltpu.SemaphoreType.DMA((2,)),
            pltpu.SemaphoreType.DMA((2,)),
        ],
    )
    def gather_kernel(table_hbm, qT_hbm, tT_hbm, out_q, out_t,
                      idx_v, gbuf, tbuf, gsem, wsem):
        c = lax.axis_index("c")
        s = lax.axis_index("s")
        wid = s * NC + c
        col0 = wid * LANES

        k = 0
        units = []
        for arr, out_ref, tls in ((qT_hbm, out_q, q_tiles),
                                  (tT_hbm, out_t, t_tiles)):
            for st, nv in tls:
                pltpu.sync_copy(
                    arr.at[pl.ds(8 * st, nv), pl.ds(col0, LANES)],
                    idx_v.at[k, pl.ds(0, nv)])
                for r in range(nv):
                    units.append((out_ref, 8 * st + r, k, r))
                k += 1

        tok_iota = lax.iota(jnp.int32, 16)

        def gstart(u, b):
            out_ref, sp, k, r = units[u]
            pltpu.async_copy(table_hbm.at[idx_v.at[k, r]],
                             gbuf.at[b], gsem.at[b])

        def gwait(b):
            pltpu.make_async_copy(table_hbm.at[idx_v.at[0, 0]],
                                  gbuf.at[b], gsem.at[b]).wait()

        def wstart(u, b):
            out_ref, sp, k, r = units[u]
            pltpu.async_copy(tbuf.at[b],
                             out_ref.at[sp, pl.ds(0, D), pl.ds(col0, LANES)],
                             wsem.at[b])

        def wwait(b):
            pltpu.make_async_copy(tbuf.at[b],
                                  out_q.at[0, pl.ds(0, D), pl.ds(0, LANES)],
                                  wsem.at[b]).wait()

        def transpose(b):
            def row(d, _):
                for tg in range(8):
                    x = plsc.load_gather(
                        gbuf.at[b],
                        [tok_iota + tg * 16, jnp.full((16,), d, jnp.int32)])
                    tbuf[b, d, pl.ds(tg * 16, 16)] = x
                return 0
            lax.fori_loop(0, D, row, 0, unroll=False)

        n_units = len(units)
        gstart(0, 0)
        for u in range(n_units):
            b = u % 2
            gwait(b)
            if u + 1 < n_units:
                gstart(u + 1, b ^ 1)
            if u >= 2:
                wwait(b)
            transpose(b)
            wstart(u, b)
        wwait((n_units - 1) % 2)
        wwait(n_units % 2)

    return gather_kernel


def kernel(table, query, title):
    V, D = table.shape
    B, SQ = query.shape
    _, ST = title.shape
    table128 = jnp.pad(table, ((0, 0), (0, LANES - D)))
    qT = jnp.transpose(query.astype(jnp.int32))
    tT = jnp.transpose(title.astype(jnp.int32))
    fn = _make_gather(V, D, SQ, ST, B)
    out_qT, out_tT = fn(table128, qT, tT)
    return (jnp.transpose(out_qT, (2, 0, 1)), jnp.transpose(out_tT, (2, 0, 1)))
